# baseline (speedup 1.0000x reference)
import functools

import jax
import jax.numpy as jnp
from jax import lax
from jax.experimental import pallas as pl
from jax.experimental.pallas import tpu as pltpu
from jax.experimental.pallas import tpu_sc as plsc

N, D = 65536, 128
NUM_SB, NUM_LAB = 4, 64
NSEG = NUM_SB * NUM_LAB
DV, DD = 0.5, 1.5
NC, NS = 2, 16
NW = NC * NS
ROWS_W = N // NW
CHUNK = 128
NCHUNK = ROWS_W // CHUNK
DC = D // 16
NG = CHUNK // 16

_mesh = plsc.VectorSubcoreMesh(
    core_axis_name="c", subcore_axis_name="s", num_cores=NC, num_subcores=NS)


def _rsqrt(ss):
    xb = lax.bitcast_convert_type(ss, jnp.int32)
    y = lax.bitcast_convert_type(jnp.int32(0x5F3759DF) - (xb >> 1), jnp.float32)
    for _ in range(3):
        y = y * (1.5 - 0.5 * ss * y * y)
    return y


_GDN = lax.GatherDimensionNumbers(
    offset_dims=(), collapsed_slice_dims=(0,), start_index_map=(0,))


def _shuffle(v, idx):
    return lax.gather(v, idx[:, None], _GDN, (1,),
                      mode=lax.GatherScatterMode.PROMISE_IN_BOUNDS)


def _lanesum(v, lanes):
    for sh in (8, 4, 2, 1):
        v = v + _shuffle(v, lanes ^ sh)
    return v


@functools.partial(
    pl.kernel,
    out_type=(
        jax.ShapeDtypeStruct((NC, NSEG, D), jnp.float32),
        jax.ShapeDtypeStruct((NW, NSEG), jnp.float32),
        jax.ShapeDtypeStruct((N, D), jnp.float32),
    ),
    mesh=_mesh,
    scratch_types=[
        pltpu.VMEM((2, CHUNK, D), jnp.float32),
        pltpu.VMEM((2, CHUNK, D), jnp.float32),
        pltpu.VMEM((2, CHUNK), jnp.int32),
        pltpu.VMEM((2, CHUNK), jnp.int32),
        pltpu.VMEM((2, CHUNK), jnp.int32),
        pltpu.VMEM((NSEG + 16,), jnp.float32),
        pltpu.VMEM((16, D), jnp.float32),
        pltpu.VMEM_SHARED((NSEG, D), jnp.float32),
        pltpu.SemaphoreType.DMA,
        pltpu.SemaphoreType.DMA,
        pltpu.SemaphoreType.DMA,
        pltpu.SemaphoreType.DMA,
        pltpu.SemaphoreType.DMA,
        pltpu.SemaphoreType.DMA,
    ],
)
def _k1(x_hbm, lab_hbm, sb_hbm, psum_hbm, cnt_hbm, xn_hbm,
        rawbuf, nrmbuf, labbuf, sbbuf, segbuf, counts, slicebuf, cent_sh,
        in_sem0, in_sem1, sc_sem0, sc_sem1, xn_sem0, xn_sem1):
    c = lax.axis_index("c")
    s = lax.axis_index("s")
    wid = c * NS + s
    row0 = wid * ROWS_W
    zv = jnp.zeros((16,), jnp.float32)
    lanes = lax.iota(jnp.int32, 16)
    onehot0 = jnp.where(lanes == 0, 1.0, 0.0).astype(jnp.float32)
    in_sems = (in_sem0, in_sem1)
    sc_sems = (sc_sem0, sc_sem1)
    xn_sems = (xn_sem0, xn_sem1)

    def zrow(i, _):
        for j in range(DC):
            slicebuf[i, pl.ds(j * 16, 16)] = zv
        return 0
    lax.fori_loop(0, 16, zrow, 0)
    pltpu.sync_copy(slicebuf, cent_sh.at[pl.ds(s * 16, 16)])

    def zcnt(i, _):
        counts[pl.ds(i * 16, 16)] = zv
        return 0
    lax.fori_loop(0, NSEG // 16 + 1, zcnt, 0)
    plsc.subcore_barrier()

    def start_in(b, ci):
        base = row0 + ci * CHUNK
        pltpu.async_copy(x_hbm.at[pl.ds(base, CHUNK)], rawbuf.at[b], in_sems[b])
        pltpu.async_copy(lab_hbm.at[pl.ds(base, CHUNK)], labbuf.at[b], in_sems[b])
        pltpu.async_copy(sb_hbm.at[pl.ds(base, CHUNK)], sbbuf.at[b], in_sems[b])

    def wait_in(b):
        pltpu.make_async_copy(
            x_hbm.at[pl.ds(row0, CHUNK)], rawbuf.at[b], in_sems[b]).wait()
        pltpu.make_async_copy(
            lab_hbm.at[pl.ds(row0, CHUNK)], labbuf.at[b], in_sems[b]).wait()
        pltpu.make_async_copy(
            sb_hbm.at[pl.ds(row0, CHUNK)], sbbuf.at[b], in_sems[b]).wait()

    def start_scat(b, ci):
        pltpu.async_copy(nrmbuf.at[b], cent_sh.at[segbuf.at[b]],
                         sc_sems[b], add=True)
        base = row0 + ci * CHUNK
        pltpu.async_copy(nrmbuf.at[b], xn_hbm.at[pl.ds(base, CHUNK)],
                         xn_sems[b])

    def wait_scat(b):
        pltpu.make_async_copy(
            nrmbuf.at[b], cent_sh.at[segbuf.at[b]], sc_sems[b]).wait()
        pltpu.make_async_copy(
            nrmbuf.at[b], xn_hbm.at[pl.ds(row0, CHUNK)], xn_sems[b]).wait()

    def compute(b):
        def grp(g, _):
            gbase = g * 16
            lv = labbuf[b, pl.ds(gbase, 16)]
            sv = sbbuf[b, pl.ds(gbase, 16)]
            segv = sv * NUM_LAB + lv
            segbuf[b, pl.ds(gbase, 16)] = segv
            for k in range(16):
                r = gbase + k
                ssv = jnp.zeros((16,), jnp.float32)
                xs = []
                for j in range(DC):
                    xj = rawbuf[b, r, pl.ds(j * 16, 16)]
                    xs.append(xj)
                    ssv = ssv + xj * xj
                ss = jnp.maximum(_lanesum(ssv, lanes), 1e-30)
                factor = _rsqrt(ss)
                for j in range(DC):
                    nrmbuf[b, r, pl.ds(j * 16, 16)] = xs[j] * factor
                sg = segv[k]
                cv = counts[pl.ds(sg, 16)]
                counts[pl.ds(sg, 16)] = cv + onehot0
            return 0
        lax.fori_loop(0, NG, grp, 0)

    start_in(0, 0)
    start_in(1, 1)
    for b in range(2):
        wait_in(b)
        compute(b)
        start_scat(b, b)
        start_in(b, 2 + b)

    def pair_body(i, _):
        ci0 = 2 * i
        for b in range(2):
            wait_in(b)
            wait_scat(b)
            compute(b)
            start_scat(b, ci0 + b)
            start_in(b, jnp.minimum(ci0 + b + 2, NCHUNK - 2 + b))
        return 0
    lax.fori_loop(1, NCHUNK // 2, pair_body, 0)
    wait_in(0)
    wait_in(1)
    wait_scat(0)
    wait_scat(1)

    plsc.subcore_barrier()
    pltpu.sync_copy(cent_sh.at[pl.ds(s * 16, 16)], slicebuf)
    pltpu.sync_copy(slicebuf, psum_hbm.at[c, pl.ds(s * 16, 16)])
    pltpu.sync_copy(counts.at[pl.ds(0, NSEG)], cnt_hbm.at[wid])


def _k2_body(psum_ref, cnt_ref, mu_ref, w_ref, push_ref):
    counts = jnp.sum(cnt_ref[...], axis=0)
    sums = psum_ref[0] + psum_ref[1]
    safe_cnt = jnp.maximum(counts, 1.0)
    mu = sums / safe_cnt
    mu_ref[...] = mu
    seg_iota = lax.broadcasted_iota(jnp.int32, (NSEG, 1), 0)
    sb_col = seg_iota // NUM_LAB
    presentf = (counts > 0.0).astype(jnp.float32)
    Mf = []
    any_pts = []
    for sb in range(NUM_SB):
        msk = (sb_col == sb).astype(jnp.float32)
        Mf.append(jnp.sum(presentf * msk))
        any_pts.append((jnp.sum(counts * msk) > 0.0).astype(jnp.float32))
    B = any_pts[0] + any_pts[1] + any_pts[2] + any_pts[3]
    safe_B = jnp.maximum(B, 1.0)
    w_col = jnp.zeros((NSEG, 1), jnp.float32)
    scale_col = jnp.zeros((NSEG, 1), jnp.float32)
    for sb in range(NUM_SB):
        msk = (sb_col == sb).astype(jnp.float32)
        active = (Mf[sb] > 1.0).astype(jnp.float32)
        w_sb = active / (jnp.maximum(Mf[sb], 1.0) * safe_B)
        w_col = w_col + msk * w_sb / safe_cnt
        s_sb = active / (safe_B * jnp.maximum(Mf[sb] * (Mf[sb] - 1.0), 1.0))
        scale_col = scale_col + msk * s_sb
    w_ref[...] = w_col
    scale_col = scale_col * presentf

    def body(j, acc):
        rowj = mu_ref[pl.ds(j, 1), :]
        pd = jnp.sum(jnp.abs(mu_ref[...] - rowj), axis=1, keepdims=True)
        sbj = j // NUM_LAB
        m = ((sb_col == sbj) & (seg_iota != j)).astype(jnp.float32)
        push = jnp.maximum(2.0 * DD - pd, 0.0)
        pj = jnp.sum(jnp.where(seg_iota == j, presentf, 0.0))
        return acc + jnp.sum(push * push * m * scale_col) * pj
    total = lax.fori_loop(0, NSEG, body, jnp.float32(0.0))
    push_ref[...] = jnp.full((1, 1), total, jnp.float32)


_k2 = pl.pallas_call(
    _k2_body,
    out_shape=(
        jax.ShapeDtypeStruct((NSEG, D), jnp.float32),
        jax.ShapeDtypeStruct((NSEG, 1), jnp.float32),
        jax.ShapeDtypeStruct((1, 1), jnp.float32),
    ),
)


@functools.partial(
    pl.kernel,
    out_type=jax.ShapeDtypeStruct((NW, 16), jnp.float32),
    mesh=_mesh,
    scratch_types=[
        pltpu.VMEM((NSEG, D), jnp.float32),
        pltpu.VMEM((NSEG + 16,), jnp.float32),
        pltpu.VMEM((2, CHUNK, D), jnp.float32),
        pltpu.VMEM((2, CHUNK), jnp.int32),
        pltpu.VMEM((2, CHUNK), jnp.int32),
        pltpu.VMEM((16,), jnp.float32),
        pltpu.SemaphoreType.DMA,
        pltpu.SemaphoreType.DMA,
    ],
)
def _k3(x_hbm, lab_hbm, sb_hbm, mu_hbm, w_hbm, part_hbm,
        mubuf, wbuf, rowbuf, labbuf, sbbuf, outvec, in_sem0, in_sem1):
    c = lax.axis_index("c")
    s = lax.axis_index("s")
    wid = c * NS + s
    row0 = wid * ROWS_W
    pltpu.sync_copy(mu_hbm, mubuf)
    pltpu.sync_copy(w_hbm, wbuf.at[pl.ds(0, NSEG)])
    lanes = lax.iota(jnp.int32, 16)
    in_sems = (in_sem0, in_sem1)

    def start_in(b, ci):
        base = row0 + ci * CHUNK
        pltpu.async_copy(x_hbm.at[pl.ds(base, CHUNK)], rowbuf.at[b], in_sems[b])
        pltpu.async_copy(lab_hbm.at[pl.ds(base, CHUNK)], labbuf.at[b], in_sems[b])
        pltpu.async_copy(sb_hbm.at[pl.ds(base, CHUNK)], sbbuf.at[b], in_sems[b])

    def wait_in(b):
        pltpu.make_async_copy(
            x_hbm.at[pl.ds(row0, CHUNK)], rowbuf.at[b], in_sems[b]).wait()
        pltpu.make_async_copy(
            lab_hbm.at[pl.ds(row0, CHUNK)], labbuf.at[b], in_sems[b]).wait()
        pltpu.make_async_copy(
            sb_hbm.at[pl.ds(row0, CHUNK)], sbbuf.at[b], in_sems[b]).wait()

    def compute(b, acc):
        def grp(g, a):
            gbase = g * 16
            lv = labbuf[b, pl.ds(gbase, 16)]
            sv = sbbuf[b, pl.ds(gbase, 16)]
            segv = sv * NUM_LAB + lv
            for k in range(16):
                r = gbase + k
                sg = segv[k]
                wk = wbuf[pl.ds(sg, 16)][0]
                dv = jnp.zeros((16,), jnp.float32)
                for j in range(DC):
                    muj = mubuf[sg, pl.ds(j * 16, 16)]
                    xj = rowbuf[b, r, pl.ds(j * 16, 16)]
                    dv = dv + jnp.abs(muj - xj)
                d = _lanesum(dv, lanes)
                t = jnp.maximum(d - DV, 0.0)
                a = a + jnp.where(lanes == k, t * t * wk, 0.0)
            return a
        return lax.fori_loop(0, NG, grp, acc)

    start_in(0, 0)
    start_in(1, 1)

    def pair_body(i, acc):
        ci0 = 2 * i
        for b in range(2):
            wait_in(b)
            acc = compute(b, acc)
            start_in(b, jnp.minimum(ci0 + b + 2, NCHUNK - 2 + b))
        return acc
    acc = lax.fori_loop(0, NCHUNK // 2, pair_body,
                        jnp.zeros((16,), jnp.float32))
    wait_in(0)
    wait_in(1)
    outvec[...] = acc
    pltpu.sync_copy(outvec, part_hbm.at[wid])


def _k4_body(part_ref, push_ref, out_ref):
    out_ref[...] = jnp.full((1, 1), jnp.sum(part_ref[...]), jnp.float32) + push_ref[...]


_k4 = pl.pallas_call(
    _k4_body,
    out_shape=jax.ShapeDtypeStruct((1, 1), jnp.float32),
)


def kernel(outputs, labels, subbatch_indices):
    labels = labels.astype(jnp.int32)
    subbatch_indices = subbatch_indices.astype(jnp.int32)
    psum, cnt, xn = _k1(outputs, labels, subbatch_indices)
    mu, wtab, push = _k2(psum, cnt.reshape(NW, NSEG, 1))
    part = _k3(xn, labels, subbatch_indices, mu, wtab.reshape(NSEG))
    loss = _k4(part, push)
    return jnp.reshape(loss, ())


# speedup vs baseline: 1.3111x; 1.3111x over previous
import functools

import jax
import jax.numpy as jnp
from jax import lax
from jax.experimental import pallas as pl
from jax.experimental.pallas import tpu as pltpu
from jax.experimental.pallas import tpu_sc as plsc

N, D = 65536, 128
NUM_SB, NUM_LAB = 4, 64
NSEG = NUM_SB * NUM_LAB
DV, DD = 0.5, 1.5
NC, NS = 2, 16
NW = NC * NS
ROWS_W = N // NW
CHUNK = 128
NCHUNK = ROWS_W // CHUNK
DC = D // 16
NG = CHUNK // 16

_mesh = plsc.VectorSubcoreMesh(
    core_axis_name="c", subcore_axis_name="s", num_cores=NC, num_subcores=NS)


def _rsqrt(ss):
    xb = lax.bitcast_convert_type(ss, jnp.int32)
    y = lax.bitcast_convert_type(jnp.int32(0x5F3759DF) - (xb >> 1), jnp.float32)
    for _ in range(3):
        y = y * (1.5 - 0.5 * ss * y * y)
    return y


_GDN = lax.GatherDimensionNumbers(
    offset_dims=(), collapsed_slice_dims=(0,), start_index_map=(0,))


def _shuffle(v, idx):
    return lax.gather(v, idx[:, None], _GDN, (1,),
                      mode=lax.GatherScatterMode.PROMISE_IN_BOUNDS)


def _lanesum(v, lanes):
    for sh in (8, 4, 2, 1):
        v = v + _shuffle(v, lanes ^ sh)
    return v


@functools.partial(
    pl.kernel,
    out_type=(
        jax.ShapeDtypeStruct((NC, NSEG, D), jnp.float32),
        jax.ShapeDtypeStruct((NW, NSEG), jnp.float32),
    ),
    mesh=_mesh,
    scratch_types=[
        pltpu.VMEM((2, CHUNK, D), jnp.float32),
        pltpu.VMEM((2, CHUNK, D), jnp.float32),
        pltpu.VMEM((2, CHUNK), jnp.int32),
        pltpu.VMEM((2, CHUNK), jnp.int32),
        pltpu.VMEM((2, CHUNK), jnp.int32),
        pltpu.VMEM((NSEG + 16,), jnp.float32),
        pltpu.VMEM((16, D), jnp.float32),
        pltpu.VMEM_SHARED((NSEG, D), jnp.float32),
        pltpu.SemaphoreType.DMA,
        pltpu.SemaphoreType.DMA,
        pltpu.SemaphoreType.DMA,
        pltpu.SemaphoreType.DMA,
    ],
)
def _k1(x_hbm, lab_hbm, sb_hbm, psum_hbm, cnt_hbm,
        rawbuf, nrmbuf, labbuf, sbbuf, segbuf, counts, slicebuf, cent_sh,
        in_sem0, in_sem1, sc_sem0, sc_sem1):
    c = lax.axis_index("c")
    s = lax.axis_index("s")
    wid = c * NS + s
    row0 = wid * ROWS_W
    zv = jnp.zeros((16,), jnp.float32)
    lanes = lax.iota(jnp.int32, 16)
    onehot0 = jnp.where(lanes == 0, 1.0, 0.0).astype(jnp.float32)
    in_sems = (in_sem0, in_sem1)
    sc_sems = (sc_sem0, sc_sem1)

    def zrow(i, _):
        for j in range(DC):
            slicebuf[i, pl.ds(j * 16, 16)] = zv
        return 0
    lax.fori_loop(0, 16, zrow, 0)
    pltpu.sync_copy(slicebuf, cent_sh.at[pl.ds(s * 16, 16)])

    def zcnt(i, _):
        counts[pl.ds(i * 16, 16)] = zv
        return 0
    lax.fori_loop(0, NSEG // 16 + 1, zcnt, 0)
    plsc.subcore_barrier()

    def start_in(b, ci):
        base = row0 + ci * CHUNK
        pltpu.async_copy(x_hbm.at[pl.ds(base, CHUNK)], rawbuf.at[b], in_sems[b])
        pltpu.async_copy(lab_hbm.at[pl.ds(base, CHUNK)], labbuf.at[b], in_sems[b])
        pltpu.async_copy(sb_hbm.at[pl.ds(base, CHUNK)], sbbuf.at[b], in_sems[b])

    def wait_in(b):
        pltpu.make_async_copy(
            x_hbm.at[pl.ds(row0, CHUNK)], rawbuf.at[b], in_sems[b]).wait()
        pltpu.make_async_copy(
            lab_hbm.at[pl.ds(row0, CHUNK)], labbuf.at[b], in_sems[b]).wait()
        pltpu.make_async_copy(
            sb_hbm.at[pl.ds(row0, CHUNK)], sbbuf.at[b], in_sems[b]).wait()

    def start_scat(b, ci):
        pltpu.async_copy(nrmbuf.at[b], cent_sh.at[segbuf.at[b]],
                         sc_sems[b], add=True)

    def wait_scat(b):
        pltpu.make_async_copy(
            nrmbuf.at[b], cent_sh.at[segbuf.at[b]], sc_sems[b]).wait()

    def compute(b):
        def grp(g, _):
            gbase = g * 16
            lv = labbuf[b, pl.ds(gbase, 16)]
            sv = sbbuf[b, pl.ds(gbase, 16)]
            segv = sv * NUM_LAB + lv
            segbuf[b, pl.ds(gbase, 16)] = segv
            for k in range(16):
                r = gbase + k
                ssv = jnp.zeros((16,), jnp.float32)
                xs = []
                for j in range(DC):
                    xj = rawbuf[b, r, pl.ds(j * 16, 16)]
                    xs.append(xj)
                    ssv = ssv + xj * xj
                ss = jnp.maximum(_lanesum(ssv, lanes), 1e-30)
                factor = _rsqrt(ss)
                for j in range(DC):
                    nrmbuf[b, r, pl.ds(j * 16, 16)] = xs[j] * factor
                sg = segv[k]
                cv = counts[pl.ds(sg, 16)]
                counts[pl.ds(sg, 16)] = cv + onehot0
            return 0
        lax.fori_loop(0, NG, grp, 0)

    start_in(0, 0)
    start_in(1, 1)
    for b in range(2):
        wait_in(b)
        compute(b)
        start_scat(b, b)
        start_in(b, 2 + b)

    def pair_body(i, _):
        ci0 = 2 * i
        for b in range(2):
            wait_in(b)
            wait_scat(b)
            compute(b)
            start_scat(b, ci0 + b)
            start_in(b, jnp.minimum(ci0 + b + 2, NCHUNK - 2 + b))
        return 0
    lax.fori_loop(1, NCHUNK // 2, pair_body, 0)
    wait_in(0)
    wait_in(1)
    wait_scat(0)
    wait_scat(1)

    plsc.subcore_barrier()
    pltpu.sync_copy(cent_sh.at[pl.ds(s * 16, 16)], slicebuf)
    pltpu.sync_copy(slicebuf, psum_hbm.at[c, pl.ds(s * 16, 16)])
    pltpu.sync_copy(counts.at[pl.ds(0, NSEG)], cnt_hbm.at[wid])


def _k2_body(psum_ref, cnt_ref, mu_ref, w_ref, push_ref):
    counts = jnp.sum(cnt_ref[...], axis=0)
    sums = psum_ref[0] + psum_ref[1]
    safe_cnt = jnp.maximum(counts, 1.0)
    mu = sums / safe_cnt
    mu_ref[...] = mu
    seg_iota = lax.broadcasted_iota(jnp.int32, (NSEG, 1), 0)
    sb_col = seg_iota // NUM_LAB
    presentf = (counts > 0.0).astype(jnp.float32)
    Mf = []
    any_pts = []
    for sb in range(NUM_SB):
        msk = (sb_col == sb).astype(jnp.float32)
        Mf.append(jnp.sum(presentf * msk))
        any_pts.append((jnp.sum(counts * msk) > 0.0).astype(jnp.float32))
    B = any_pts[0] + any_pts[1] + any_pts[2] + any_pts[3]
    safe_B = jnp.maximum(B, 1.0)
    w_col = jnp.zeros((NSEG, 1), jnp.float32)
    scale_col = jnp.zeros((NSEG, 1), jnp.float32)
    for sb in range(NUM_SB):
        msk = (sb_col == sb).astype(jnp.float32)
        active = (Mf[sb] > 1.0).astype(jnp.float32)
        w_sb = active / (jnp.maximum(Mf[sb], 1.0) * safe_B)
        w_col = w_col + msk * w_sb / safe_cnt
        s_sb = active / (safe_B * jnp.maximum(Mf[sb] * (Mf[sb] - 1.0), 1.0))
        scale_col = scale_col + msk * s_sb
    w_ref[...] = w_col
    scale_col = scale_col * presentf

    def body(j, acc):
        rowj = mu_ref[pl.ds(j, 1), :]
        pd = jnp.sum(jnp.abs(mu_ref[...] - rowj), axis=1, keepdims=True)
        sbj = j // NUM_LAB
        m = ((sb_col == sbj) & (seg_iota != j)).astype(jnp.float32)
        push = jnp.maximum(2.0 * DD - pd, 0.0)
        pj = jnp.sum(jnp.where(seg_iota == j, presentf, 0.0))
        return acc + jnp.sum(push * push * m * scale_col) * pj
    total = lax.fori_loop(0, NSEG, body, jnp.float32(0.0))
    push_ref[...] = jnp.full((1, 1), total, jnp.float32)


_k2 = pl.pallas_call(
    _k2_body,
    out_shape=(
        jax.ShapeDtypeStruct((NSEG, D), jnp.float32),
        jax.ShapeDtypeStruct((NSEG, 1), jnp.float32),
        jax.ShapeDtypeStruct((1, 1), jnp.float32),
    ),
)


@functools.partial(
    pl.kernel,
    out_type=jax.ShapeDtypeStruct((NW, 16), jnp.float32),
    mesh=_mesh,
    scratch_types=[
        pltpu.VMEM((NSEG, D), jnp.float32),
        pltpu.VMEM((NSEG + 16,), jnp.float32),
        pltpu.VMEM((2, CHUNK, D), jnp.float32),
        pltpu.VMEM((2, CHUNK), jnp.int32),
        pltpu.VMEM((2, CHUNK), jnp.int32),
        pltpu.VMEM((16,), jnp.float32),
        pltpu.SemaphoreType.DMA,
        pltpu.SemaphoreType.DMA,
    ],
)
def _k3(x_hbm, lab_hbm, sb_hbm, mu_hbm, w_hbm, part_hbm,
        mubuf, wbuf, rowbuf, labbuf, sbbuf, outvec, in_sem0, in_sem1):
    c = lax.axis_index("c")
    s = lax.axis_index("s")
    wid = c * NS + s
    row0 = wid * ROWS_W
    pltpu.sync_copy(mu_hbm, mubuf)
    pltpu.sync_copy(w_hbm, wbuf.at[pl.ds(0, NSEG)])
    lanes = lax.iota(jnp.int32, 16)
    in_sems = (in_sem0, in_sem1)

    def start_in(b, ci):
        base = row0 + ci * CHUNK
        pltpu.async_copy(x_hbm.at[pl.ds(base, CHUNK)], rowbuf.at[b], in_sems[b])
        pltpu.async_copy(lab_hbm.at[pl.ds(base, CHUNK)], labbuf.at[b], in_sems[b])
        pltpu.async_copy(sb_hbm.at[pl.ds(base, CHUNK)], sbbuf.at[b], in_sems[b])

    def wait_in(b):
        pltpu.make_async_copy(
            x_hbm.at[pl.ds(row0, CHUNK)], rowbuf.at[b], in_sems[b]).wait()
        pltpu.make_async_copy(
            lab_hbm.at[pl.ds(row0, CHUNK)], labbuf.at[b], in_sems[b]).wait()
        pltpu.make_async_copy(
            sb_hbm.at[pl.ds(row0, CHUNK)], sbbuf.at[b], in_sems[b]).wait()

    def compute(b, acc):
        def grp(g, a):
            gbase = g * 16
            lv = labbuf[b, pl.ds(gbase, 16)]
            sv = sbbuf[b, pl.ds(gbase, 16)]
            segv = sv * NUM_LAB + lv
            for k in range(16):
                r = gbase + k
                sg = segv[k]
                wk = wbuf[pl.ds(sg, 16)][0]
                ssv = jnp.zeros((16,), jnp.float32)
                xs = []
                for j in range(DC):
                    xj = rowbuf[b, r, pl.ds(j * 16, 16)]
                    xs.append(xj)
                    ssv = ssv + xj * xj
                factor = _rsqrt(jnp.maximum(_lanesum(ssv, lanes), 1e-30))
                dv = jnp.zeros((16,), jnp.float32)
                for j in range(DC):
                    muj = mubuf[sg, pl.ds(j * 16, 16)]
                    dv = dv + jnp.abs(muj - xs[j] * factor)
                d = _lanesum(dv, lanes)
                t = jnp.maximum(d - DV, 0.0)
                a = a + jnp.where(lanes == k, t * t * wk, 0.0)
            return a
        return lax.fori_loop(0, NG, grp, acc)

    start_in(0, 0)
    start_in(1, 1)

    def pair_body(i, acc):
        ci0 = 2 * i
        for b in range(2):
            wait_in(b)
            acc = compute(b, acc)
            start_in(b, jnp.minimum(ci0 + b + 2, NCHUNK - 2 + b))
        return acc
    acc = lax.fori_loop(0, NCHUNK // 2, pair_body,
                        jnp.zeros((16,), jnp.float32))
    wait_in(0)
    wait_in(1)
    outvec[...] = acc
    pltpu.sync_copy(outvec, part_hbm.at[wid])


def _k4_body(part_ref, push_ref, out_ref):
    out_ref[...] = jnp.full((1, 1), jnp.sum(part_ref[...]), jnp.float32) + push_ref[...]


_k4 = pl.pallas_call(
    _k4_body,
    out_shape=jax.ShapeDtypeStruct((1, 1), jnp.float32),
)


def kernel(outputs, labels, subbatch_indices):
    labels = labels.astype(jnp.int32)
    subbatch_indices = subbatch_indices.astype(jnp.int32)
    psum, cnt = _k1(outputs, labels, subbatch_indices)
    mu, wtab, push = _k2(psum, cnt.reshape(NW, NSEG, 1))
    part = _k3(outputs, labels, subbatch_indices, mu, wtab.reshape(NSEG))
    loss = _k4(part, push)
    return jnp.reshape(loss, ())


# speedup vs baseline: 1.3763x; 1.0497x over previous
import functools

import jax
import jax.numpy as jnp
from jax import lax
from jax.experimental import pallas as pl
from jax.experimental.pallas import tpu as pltpu
from jax.experimental.pallas import tpu_sc as plsc

N, D = 65536, 128
NUM_SB, NUM_LAB = 4, 64
NSEG = NUM_SB * NUM_LAB
DV, DD = 0.5, 1.5
NC, NS = 2, 16
NW = NC * NS
ROWS_W = N // NW
CHUNK = 128
NCHUNK = ROWS_W // CHUNK
DC = D // 16
NG = CHUNK // 16

_mesh = plsc.VectorSubcoreMesh(
    core_axis_name="c", subcore_axis_name="s", num_cores=NC, num_subcores=NS)


def _rsqrt(ss):
    xb = lax.bitcast_convert_type(ss, jnp.int32)
    y = lax.bitcast_convert_type(jnp.int32(0x5F3759DF) - (xb >> 1), jnp.float32)
    for _ in range(2):
        y = y * (1.5 - 0.5 * ss * y * y)
    return y


_GDN = lax.GatherDimensionNumbers(
    offset_dims=(), collapsed_slice_dims=(0,), start_index_map=(0,))


def _shuffle(v, idx):
    return lax.gather(v, idx[:, None], _GDN, (1,),
                      mode=lax.GatherScatterMode.PROMISE_IN_BOUNDS)


def _lanesum(v, lanes):
    for sh in (8, 4, 2, 1):
        v = v + _shuffle(v, lanes ^ sh)
    return v


@functools.partial(
    pl.kernel,
    out_type=(
        jax.ShapeDtypeStruct((NC, NSEG, D), jnp.float32),
        jax.ShapeDtypeStruct((NW, NSEG), jnp.float32),
    ),
    mesh=_mesh,
    scratch_types=[
        pltpu.VMEM((2, CHUNK, D), jnp.float32),
        pltpu.VMEM((2, CHUNK, D), jnp.float32),
        pltpu.VMEM((2, CHUNK), jnp.int32),
        pltpu.VMEM((2, CHUNK), jnp.int32),
        pltpu.VMEM((2, CHUNK), jnp.int32),
        pltpu.VMEM((NSEG + 16,), jnp.float32),
        pltpu.VMEM((16, D), jnp.float32),
        pltpu.VMEM_SHARED((NSEG, D), jnp.float32),
        pltpu.SemaphoreType.DMA,
        pltpu.SemaphoreType.DMA,
        pltpu.SemaphoreType.DMA,
        pltpu.SemaphoreType.DMA,
    ],
)
def _k1(x_hbm, lab_hbm, sb_hbm, psum_hbm, cnt_hbm,
        rawbuf, nrmbuf, labbuf, sbbuf, segbuf, counts, slicebuf, cent_sh,
        in_sem0, in_sem1, sc_sem0, sc_sem1):
    c = lax.axis_index("c")
    s = lax.axis_index("s")
    wid = c * NS + s
    row0 = wid * ROWS_W
    zv = jnp.zeros((16,), jnp.float32)
    lanes = lax.iota(jnp.int32, 16)
    onehot0 = jnp.where(lanes == 0, 1.0, 0.0).astype(jnp.float32)
    in_sems = (in_sem0, in_sem1)
    sc_sems = (sc_sem0, sc_sem1)

    def zrow(i, _):
        for j in range(DC):
            slicebuf[i, pl.ds(j * 16, 16)] = zv
        return 0
    lax.fori_loop(0, 16, zrow, 0)
    pltpu.sync_copy(slicebuf, cent_sh.at[pl.ds(s * 16, 16)])

    def zcnt(i, _):
        counts[pl.ds(i * 16, 16)] = zv
        return 0
    lax.fori_loop(0, NSEG // 16 + 1, zcnt, 0)
    plsc.subcore_barrier()

    def start_in(b, ci):
        base = row0 + ci * CHUNK
        pltpu.async_copy(x_hbm.at[pl.ds(base, CHUNK)], rawbuf.at[b], in_sems[b])
        pltpu.async_copy(lab_hbm.at[pl.ds(base, CHUNK)], labbuf.at[b], in_sems[b])
        pltpu.async_copy(sb_hbm.at[pl.ds(base, CHUNK)], sbbuf.at[b], in_sems[b])

    def wait_in(b):
        pltpu.make_async_copy(
            x_hbm.at[pl.ds(row0, CHUNK)], rawbuf.at[b], in_sems[b]).wait()
        pltpu.make_async_copy(
            lab_hbm.at[pl.ds(row0, CHUNK)], labbuf.at[b], in_sems[b]).wait()
        pltpu.make_async_copy(
            sb_hbm.at[pl.ds(row0, CHUNK)], sbbuf.at[b], in_sems[b]).wait()

    def start_scat(b, ci):
        pltpu.async_copy(nrmbuf.at[b], cent_sh.at[segbuf.at[b]],
                         sc_sems[b], add=True)

    def wait_scat(b):
        pltpu.make_async_copy(
            nrmbuf.at[b], cent_sh.at[segbuf.at[b]], sc_sems[b]).wait()

    def compute(b):
        def grp(g, _):
            gbase = g * 16
            lv = labbuf[b, pl.ds(gbase, 16)]
            sv = sbbuf[b, pl.ds(gbase, 16)]
            segv = sv * NUM_LAB + lv
            segbuf[b, pl.ds(gbase, 16)] = segv
            for k in range(16):
                r = gbase + k
                ssv = jnp.zeros((16,), jnp.float32)
                xs = []
                for j in range(DC):
                    xj = rawbuf[b, r, pl.ds(j * 16, 16)]
                    xs.append(xj)
                    ssv = ssv + xj * xj
                ss = jnp.maximum(_lanesum(ssv, lanes), 1e-30)
                factor = _rsqrt(ss)
                for j in range(DC):
                    nrmbuf[b, r, pl.ds(j * 16, 16)] = xs[j] * factor
                sg = segv[k]
                cv = counts[pl.ds(sg, 16)]
                counts[pl.ds(sg, 16)] = cv + onehot0
            return 0
        lax.fori_loop(0, NG, grp, 0)

    start_in(0, 0)
    start_in(1, 1)
    for b in range(2):
        wait_in(b)
        compute(b)
        start_scat(b, b)
        start_in(b, 2 + b)

    def pair_body(i, _):
        ci0 = 2 * i
        for b in range(2):
            wait_in(b)
            wait_scat(b)
            compute(b)
            start_scat(b, ci0 + b)
            start_in(b, jnp.minimum(ci0 + b + 2, NCHUNK - 2 + b))
        return 0
    lax.fori_loop(1, NCHUNK // 2, pair_body, 0)
    wait_in(0)
    wait_in(1)
    wait_scat(0)
    wait_scat(1)

    plsc.subcore_barrier()
    pltpu.sync_copy(cent_sh.at[pl.ds(s * 16, 16)], slicebuf)
    pltpu.sync_copy(slicebuf, psum_hbm.at[c, pl.ds(s * 16, 16)])
    pltpu.sync_copy(counts.at[pl.ds(0, NSEG)], cnt_hbm.at[wid])


def _k2_body(psum_ref, cnt_ref, mu_ref, w_ref, push_ref):
    counts = jnp.sum(cnt_ref[...], axis=0)
    sums = psum_ref[0] + psum_ref[1]
    safe_cnt = jnp.maximum(counts, 1.0)
    mu = sums / safe_cnt
    mu_ref[...] = mu
    seg_iota = lax.broadcasted_iota(jnp.int32, (NSEG, 1), 0)
    sb_col = seg_iota // NUM_LAB
    presentf = (counts > 0.0).astype(jnp.float32)
    Mf = []
    any_pts = []
    for sb in range(NUM_SB):
        msk = (sb_col == sb).astype(jnp.float32)
        Mf.append(jnp.sum(presentf * msk))
        any_pts.append((jnp.sum(counts * msk) > 0.0).astype(jnp.float32))
    B = any_pts[0] + any_pts[1] + any_pts[2] + any_pts[3]
    safe_B = jnp.maximum(B, 1.0)
    w_col = jnp.zeros((NSEG, 1), jnp.float32)
    scale_col = jnp.zeros((NSEG, 1), jnp.float32)
    for sb in range(NUM_SB):
        msk = (sb_col == sb).astype(jnp.float32)
        active = (Mf[sb] > 1.0).astype(jnp.float32)
        w_sb = active / (jnp.maximum(Mf[sb], 1.0) * safe_B)
        w_col = w_col + msk * w_sb / safe_cnt
        s_sb = active / (safe_B * jnp.maximum(Mf[sb] * (Mf[sb] - 1.0), 1.0))
        scale_col = scale_col + msk * s_sb
    w_ref[...] = w_col
    scale_col = scale_col * presentf

    def body(j, acc):
        rowj = mu_ref[pl.ds(j, 1), :]
        pd = jnp.sum(jnp.abs(mu_ref[...] - rowj), axis=1, keepdims=True)
        sbj = j // NUM_LAB
        m = ((sb_col == sbj) & (seg_iota != j)).astype(jnp.float32)
        push = jnp.maximum(2.0 * DD - pd, 0.0)
        pj = jnp.sum(jnp.where(seg_iota == j, presentf, 0.0))
        return acc + jnp.sum(push * push * m * scale_col) * pj
    total = lax.fori_loop(0, NSEG, body, jnp.float32(0.0))
    push_ref[...] = jnp.full((1, 1), total, jnp.float32)


_k2 = pl.pallas_call(
    _k2_body,
    out_shape=(
        jax.ShapeDtypeStruct((NSEG, D), jnp.float32),
        jax.ShapeDtypeStruct((NSEG, 1), jnp.float32),
        jax.ShapeDtypeStruct((1, 1), jnp.float32),
    ),
)


@functools.partial(
    pl.kernel,
    out_type=jax.ShapeDtypeStruct((NW, 16), jnp.float32),
    mesh=_mesh,
    scratch_types=[
        pltpu.VMEM((NSEG, D), jnp.float32),
        pltpu.VMEM((NSEG + 16,), jnp.float32),
        pltpu.VMEM((2, CHUNK, D), jnp.float32),
        pltpu.VMEM((2, CHUNK), jnp.int32),
        pltpu.VMEM((2, CHUNK), jnp.int32),
        pltpu.VMEM((16,), jnp.float32),
        pltpu.SemaphoreType.DMA,
        pltpu.SemaphoreType.DMA,
    ],
)
def _k3(x_hbm, lab_hbm, sb_hbm, mu_hbm, w_hbm, part_hbm,
        mubuf, wbuf, rowbuf, labbuf, sbbuf, outvec, in_sem0, in_sem1):
    c = lax.axis_index("c")
    s = lax.axis_index("s")
    wid = c * NS + s
    row0 = wid * ROWS_W
    pltpu.sync_copy(mu_hbm, mubuf)
    pltpu.sync_copy(w_hbm, wbuf.at[pl.ds(0, NSEG)])
    lanes = lax.iota(jnp.int32, 16)
    in_sems = (in_sem0, in_sem1)

    def start_in(b, ci):
        base = row0 + ci * CHUNK
        pltpu.async_copy(x_hbm.at[pl.ds(base, CHUNK)], rowbuf.at[b], in_sems[b])
        pltpu.async_copy(lab_hbm.at[pl.ds(base, CHUNK)], labbuf.at[b], in_sems[b])
        pltpu.async_copy(sb_hbm.at[pl.ds(base, CHUNK)], sbbuf.at[b], in_sems[b])

    def wait_in(b):
        pltpu.make_async_copy(
            x_hbm.at[pl.ds(row0, CHUNK)], rowbuf.at[b], in_sems[b]).wait()
        pltpu.make_async_copy(
            lab_hbm.at[pl.ds(row0, CHUNK)], labbuf.at[b], in_sems[b]).wait()
        pltpu.make_async_copy(
            sb_hbm.at[pl.ds(row0, CHUNK)], sbbuf.at[b], in_sems[b]).wait()

    def compute(b, acc):
        def grp(g, a):
            gbase = g * 16
            lv = labbuf[b, pl.ds(gbase, 16)]
            sv = sbbuf[b, pl.ds(gbase, 16)]
            segv = sv * NUM_LAB + lv
            for k in range(16):
                r = gbase + k
                sg = segv[k]
                wk = wbuf[pl.ds(sg, 16)][0]
                ssv = jnp.zeros((16,), jnp.float32)
                xs = []
                for j in range(DC):
                    xj = rowbuf[b, r, pl.ds(j * 16, 16)]
                    xs.append(xj)
                    ssv = ssv + xj * xj
                factor = _rsqrt(jnp.maximum(_lanesum(ssv, lanes), 1e-30))
                dv = jnp.zeros((16,), jnp.float32)
                for j in range(DC):
                    muj = mubuf[sg, pl.ds(j * 16, 16)]
                    dv = dv + jnp.abs(muj - xs[j] * factor)
                d = _lanesum(dv, lanes)
                t = jnp.maximum(d - DV, 0.0)
                a = a + jnp.where(lanes == k, t * t * wk, 0.0)
            return a
        return lax.fori_loop(0, NG, grp, acc)

    start_in(0, 0)
    start_in(1, 1)

    def pair_body(i, acc):
        ci0 = 2 * i
        for b in range(2):
            wait_in(b)
            acc = compute(b, acc)
            start_in(b, jnp.minimum(ci0 + b + 2, NCHUNK - 2 + b))
        return acc
    acc = lax.fori_loop(0, NCHUNK // 2, pair_body,
                        jnp.zeros((16,), jnp.float32))
    wait_in(0)
    wait_in(1)
    outvec[...] = acc
    pltpu.sync_copy(outvec, part_hbm.at[wid])


def _k4_body(part_ref, push_ref, out_ref):
    out_ref[...] = jnp.full((1, 1), jnp.sum(part_ref[...]), jnp.float32) + push_ref[...]


_k4 = pl.pallas_call(
    _k4_body,
    out_shape=jax.ShapeDtypeStruct((1, 1), jnp.float32),
)


def kernel(outputs, labels, subbatch_indices):
    labels = labels.astype(jnp.int32)
    subbatch_indices = subbatch_indices.astype(jnp.int32)
    psum, cnt = _k1(outputs, labels, subbatch_indices)
    mu, wtab, push = _k2(psum, cnt.reshape(NW, NSEG, 1))
    part = _k3(outputs, labels, subbatch_indices, mu, wtab.reshape(NSEG))
    loss = _k4(part, push)
    return jnp.reshape(loss, ())
